# baseline (speedup 1.0000x reference)
import functools

import jax
import jax.numpy as jnp
from jax import lax
from jax.experimental import pallas as pl
from jax.experimental.pallas import tpu as pltpu
from jax.experimental.pallas import tpu_sc as plsc

BATCH = 16384
HIST = 50
DIM = 64
NROW = 1000000
N = BATCH * HIST
NUM_CORES = 2
NUM_SUBCORES = 16
NW = NUM_CORES * NUM_SUBCORES
PER_W = N // NW
CHUNK = 128
NCHUNK = PER_W // CHUNK
NB = 4
TILE = 2 * DIM * 8

_TC = 16384


def _tx_body(wt_ref, out_ref):
    t = jnp.transpose(wt_ref[...], (1, 0))
    out_ref[...] = jnp.concatenate([t, t], axis=1)


_transpose_table = pl.pallas_call(
    _tx_body,
    grid=(pl.cdiv(NROW, _TC),),
    in_specs=[pl.BlockSpec((DIM, _TC), lambda i: (0, i))],
    out_specs=pl.BlockSpec((_TC, 2 * DIM), lambda i: (i, 0)),
    out_shape=jax.ShapeDtypeStruct((NROW, 2 * DIM), jnp.float32),
)

_mesh = plsc.VectorSubcoreMesh(core_axis_name="c", subcore_axis_name="s")


@functools.partial(
    pl.kernel,
    mesh=_mesh,
    out_type=jax.ShapeDtypeStruct((HIST, 8, BATCH // CHUNK, TILE), jnp.float32),
    scratch_types=[
        pltpu.VMEM((PER_W,), jnp.int32),
        pltpu.VMEM((NB, CHUNK, 2 * DIM), jnp.float32),
        pltpu.VMEM((NB, 8 * TILE), jnp.float32),
        [pltpu.SemaphoreType.DMA] * NB,
        [pltpu.SemaphoreType.DMA] * NB,
    ],
    compiler_params=pltpu.CompilerParams(
        use_tc_tiling_on_sc=False, needs_layout_passes=False),
)
def _gather_kernel(idx_hbm, table_hbm, out_hbm, idx_v, gbufs, tbufs,
                   g_sems, s_sems):
    wid = lax.axis_index("s") * NUM_CORES + lax.axis_index("c")
    base = wid * PER_W
    cg0 = wid * NCHUNK
    pltpu.sync_copy(idx_hbm.at[pl.ds(base, PER_W)], idx_v)

    iota16 = lax.broadcasted_iota(jnp.int32, (16,), 0)
    rows = [iota16 + 16 * lg for lg in range(8)]

    def start_gather(c, b):
        pltpu.async_copy(
            table_hbm.at[idx_v.at[pl.ds(c * CHUNK, CHUNK)]],
            gbufs.at[b], g_sems[b])

    def wait_gather(c, b):
        pltpu.make_async_copy(
            table_hbm.at[idx_v.at[pl.ds(c * CHUNK, CHUNK)]],
            gbufs.at[b], g_sems[b]).wait()

    def shuffle_and_store(c, b):
        cg = cg0 + c
        h = cg // (BATCH // CHUNK)
        col = cg % (BATCH // CHUNK)
        gb = gbufs.at[b]
        tb = tbufs.at[b]

        @pl.loop(0, 8)
        def _tiles(a):
            d0 = 8 * a
            o0 = a * TILE
            for s in range(8):
                cvec = jnp.broadcast_to(d0 + s, (16,)).astype(jnp.int32)
                for lg in range(8):
                    v = plsc.load_gather(gb, [rows[lg], cvec])
                    tb[pl.ds(o0 + s * 128 + lg * 16, 16)] = v
            pltpu.async_copy(
                tb.at[pl.ds(o0, TILE)], out_hbm.at[h, a, col], s_sems[b])

    def wait_stores(b):
        for a in range(8):
            pltpu.make_async_copy(
                tbufs.at[b].at[pl.ds(a * TILE, TILE)],
                out_hbm.at[0, a, 0], s_sems[b]).wait()

    for b in range(NB):
        start_gather(b, b)

    @pl.loop(0, NCHUNK // NB)
    def _group(q):
        for b in range(NB):
            c = q * NB + b

            @pl.when(c >= NB)
            def _():
                wait_stores(b)
            wait_gather(c, b)
            shuffle_and_store(c, b)

            @pl.when(c + NB < NCHUNK)
            def _():
                start_gather(c + NB, b)

    for b in range(NB):
        wait_stores(b)


def kernel(input_, weight):
    idx = input_.T.reshape(N).astype(jnp.int32)
    table = _transpose_table(weight.T)
    out5 = _gather_kernel(idx, table)
    t5 = out5.reshape(HIST, 8, BATCH // CHUNK, 8, 2 * DIM)
    return t5.transpose(2, 4, 0, 1, 3).reshape(BATCH, HIST, DIM)


# speedup vs baseline: 1.9336x; 1.9336x over previous
import functools

import jax
import jax.numpy as jnp
from jax import lax
from jax.experimental import pallas as pl
from jax.experimental.pallas import tpu as pltpu
from jax.experimental.pallas import tpu_sc as plsc

BATCH = 16384
HIST = 50
DIM = 64
NROW = 1000000
N = BATCH * HIST
NUM_CORES = 2
NUM_SUBCORES = 16
NW = NUM_CORES * NUM_SUBCORES
PER_W = N // NW
CHUNK = 400
NCHUNK = PER_W // CHUNK
NBUF = 2
NGROUP = NCHUNK // NBUF

_TC = 16384


def _eye(n, m):
    r = lax.broadcasted_iota(jnp.int32, (n, m), 0)
    c = lax.broadcasted_iota(jnp.int32, (n, m), 1)
    return jnp.where(r == c, 1.0, 0.0).astype(jnp.float32)


def _tx_body(wt_ref, out_ref):
    t = jnp.transpose(wt_ref[...], (1, 0))
    out_ref[...] = jnp.concatenate([t, t], axis=1)


_transpose_table = pl.pallas_call(
    _tx_body,
    grid=(pl.cdiv(NROW, _TC),),
    in_specs=[pl.BlockSpec((DIM, _TC), lambda i: (0, i))],
    out_specs=pl.BlockSpec((_TC, 2 * DIM), lambda i: (i, 0)),
    out_shape=jax.ShapeDtypeStruct((NROW, 2 * DIM), jnp.float32),
)

_mesh = plsc.VectorSubcoreMesh(core_axis_name="c", subcore_axis_name="s")


@functools.partial(
    pl.kernel,
    mesh=_mesh,
    out_type=jax.ShapeDtypeStruct((N, 2 * DIM), jnp.float32),
    scratch_types=[
        pltpu.VMEM((PER_W,), jnp.int32),
        pltpu.VMEM((NBUF, CHUNK, 2 * DIM), jnp.float32),
        [pltpu.SemaphoreType.DMA] * NBUF,
        [pltpu.SemaphoreType.DMA] * NBUF,
    ],
    compiler_params=pltpu.CompilerParams(use_tc_tiling_on_sc=False),
)
def _gather_kernel(idx_hbm, table_hbm, out_hbm, idx_v, bufs, g_sems, s_sems):
    wid = lax.axis_index("s") * NUM_CORES + lax.axis_index("c")
    base = wid * PER_W
    pltpu.sync_copy(idx_hbm.at[pl.ds(base, PER_W)], idx_v)

    def start_gather(c, b):
        pltpu.async_copy(
            table_hbm.at[idx_v.at[pl.ds(c * CHUNK, CHUNK)]],
            bufs.at[b], g_sems[b])

    def wait_gather(c, b):
        pltpu.make_async_copy(
            table_hbm.at[idx_v.at[pl.ds(c * CHUNK, CHUNK)]],
            bufs.at[b], g_sems[b]).wait()

    def start_store(c, b):
        pltpu.async_copy(
            bufs.at[b], out_hbm.at[pl.ds(base + c * CHUNK, CHUNK)], s_sems[b])

    def wait_store(c, b):
        pltpu.make_async_copy(
            bufs.at[b], out_hbm.at[pl.ds(base + c * CHUNK, CHUNK)],
            s_sems[b]).wait()

    for b in range(NBUF):
        start_gather(b, b)
        if b > 0:
            wait_gather(b - 1, b - 1)
            start_store(b - 1, b - 1)

    @pl.loop(1, NGROUP)
    def _group(q):
        for b in range(NBUF):
            c = q * NBUF + b
            wait_store(c - NBUF, b)
            start_gather(c, b)
            pb = (b - 1) % NBUF
            wait_gather(c - 1, pb)
            start_store(c - 1, pb)

    last = NCHUNK - 1
    lb = last % NBUF
    wait_gather(last, lb)
    start_store(last, lb)
    for b in range(NBUF):
        wait_store(NCHUNK - NBUF + b, b)


_BC = 16384


def _out_body(g_ref, out_ref):
    t = jnp.transpose(g_ref[0], (1, 0))
    out_ref[0] = t[:DIM, :]


_transpose_out = pl.pallas_call(
    _out_body,
    grid=(HIST, BATCH // _BC),
    in_specs=[pl.BlockSpec((1, _BC, 2 * DIM), lambda h, j: (h, j, 0))],
    out_specs=pl.BlockSpec((1, DIM, _BC), lambda h, j: (h, 0, j)),
    out_shape=jax.ShapeDtypeStruct((HIST, DIM, BATCH), jnp.float32),
)


def kernel(input_, weight):
    idx = input_.T.reshape(N).astype(jnp.int32)
    table = _transpose_table(weight.T)
    rows = _gather_kernel(idx, table)
    out3 = _transpose_out(rows.reshape(HIST, BATCH, 2 * DIM))
    return jnp.transpose(out3, (2, 0, 1))


# speedup vs baseline: 1.9350x; 1.0007x over previous
import functools

import jax
import jax.numpy as jnp
from jax import lax
from jax.experimental import pallas as pl
from jax.experimental.pallas import tpu as pltpu
from jax.experimental.pallas import tpu_sc as plsc

BATCH = 16384
HIST = 50
DIM = 64
NROW = 1000000
N = BATCH * HIST
NUM_CORES = 2
NUM_SUBCORES = 16
NW = NUM_CORES * NUM_SUBCORES
PER_W = N // NW
CHUNK = 200
NCHUNK = PER_W // CHUNK
NBUF = 4
NGROUP = NCHUNK // NBUF

_TC = 16384


def _eye(n, m):
    r = lax.broadcasted_iota(jnp.int32, (n, m), 0)
    c = lax.broadcasted_iota(jnp.int32, (n, m), 1)
    return jnp.where(r == c, 1.0, 0.0).astype(jnp.float32)


def _tx_body(wt_ref, out_ref):
    t = jnp.transpose(wt_ref[...], (1, 0))
    out_ref[...] = jnp.concatenate([t, t], axis=1)


_transpose_table = pl.pallas_call(
    _tx_body,
    grid=(pl.cdiv(NROW, _TC),),
    in_specs=[pl.BlockSpec((DIM, _TC), lambda i: (0, i))],
    out_specs=pl.BlockSpec((_TC, 2 * DIM), lambda i: (i, 0)),
    out_shape=jax.ShapeDtypeStruct((NROW, 2 * DIM), jnp.float32),
)

_mesh = plsc.VectorSubcoreMesh(core_axis_name="c", subcore_axis_name="s")


@functools.partial(
    pl.kernel,
    mesh=_mesh,
    out_type=jax.ShapeDtypeStruct((N, 2 * DIM), jnp.float32),
    scratch_types=[
        pltpu.VMEM((PER_W,), jnp.int32),
        pltpu.VMEM((NBUF, CHUNK, 2 * DIM), jnp.float32),
        [pltpu.SemaphoreType.DMA] * NBUF,
        [pltpu.SemaphoreType.DMA] * NBUF,
    ],
    compiler_params=pltpu.CompilerParams(use_tc_tiling_on_sc=False),
)
def _gather_kernel(idx_hbm, table_hbm, out_hbm, idx_v, bufs, g_sems, s_sems):
    wid = lax.axis_index("s") * NUM_CORES + lax.axis_index("c")
    base = wid * PER_W
    pltpu.sync_copy(idx_hbm.at[pl.ds(base, PER_W)], idx_v)

    def start_gather(c, b):
        pltpu.async_copy(
            table_hbm.at[idx_v.at[pl.ds(c * CHUNK, CHUNK)]],
            bufs.at[b], g_sems[b])

    def wait_gather(c, b):
        pltpu.make_async_copy(
            table_hbm.at[idx_v.at[pl.ds(c * CHUNK, CHUNK)]],
            bufs.at[b], g_sems[b]).wait()

    def start_store(c, b):
        pltpu.async_copy(
            bufs.at[b], out_hbm.at[pl.ds(base + c * CHUNK, CHUNK)], s_sems[b])

    def wait_store(c, b):
        pltpu.make_async_copy(
            bufs.at[b], out_hbm.at[pl.ds(base + c * CHUNK, CHUNK)],
            s_sems[b]).wait()

    for b in range(NBUF):
        start_gather(b, b)
        if b > 0:
            wait_gather(b - 1, b - 1)
            start_store(b - 1, b - 1)

    @pl.loop(1, NGROUP)
    def _group(q):
        for b in range(NBUF):
            c = q * NBUF + b
            wait_store(c - NBUF, b)
            start_gather(c, b)
            pb = (b - 1) % NBUF
            wait_gather(c - 1, pb)
            start_store(c - 1, pb)

    last = NCHUNK - 1
    lb = last % NBUF
    wait_gather(last, lb)
    start_store(last, lb)
    for b in range(NBUF):
        wait_store(NCHUNK - NBUF + b, b)


_BC = 16384


def _out_body(g_ref, out_ref):
    t = jnp.transpose(g_ref[0], (1, 0))
    out_ref[0] = t[:DIM, :]


_transpose_out = pl.pallas_call(
    _out_body,
    grid=(HIST, BATCH // _BC),
    in_specs=[pl.BlockSpec((1, _BC, 2 * DIM), lambda h, j: (h, j, 0))],
    out_specs=pl.BlockSpec((1, DIM, _BC), lambda h, j: (h, 0, j)),
    out_shape=jax.ShapeDtypeStruct((HIST, DIM, BATCH), jnp.float32),
)


def kernel(input_, weight):
    idx = input_.T.reshape(N).astype(jnp.int32)
    table = _transpose_table(weight.T)
    rows = _gather_kernel(idx, table)
    out3 = _transpose_out(rows.reshape(HIST, BATCH, 2 * DIM))
    return jnp.transpose(out3, (2, 0, 1))
